# baseline (speedup 1.0000x reference)
import functools

import jax
import jax.numpy as jnp
from jax.experimental import pallas as pl
from jax.experimental.pallas import tpu as pltpu

_F32_MAX = 3.0e38


def _dist_body(NI, NJ, TR, TC, CH,
               ref_blk, srcT_blk, gt_smem,
               fwd_min_o, fwd_idx_o, bwd_min_o, bwd_idx_o,
               A, I, bmin_s, bidx_s):
    i = pl.program_id(0)
    j = pl.program_id(1)
    NCH = TR // CH

    sx = srcT_blk[0:1, :]
    sy = srcT_blk[1:2, :]
    sz = srcT_blk[2:3, :]
    bx = gt_smem[0, 0] * sx + gt_smem[0, 1] * sy + gt_smem[0, 2] * sz + gt_smem[0, 3]
    by = gt_smem[1, 0] * sx + gt_smem[1, 1] * sy + gt_smem[1, 2] * sz + gt_smem[1, 3]
    bz = gt_smem[2, 0] * sx + gt_smem[2, 1] * sy + gt_smem[2, 2] * sz + gt_smem[2, 3]

    colf = (jax.lax.broadcasted_iota(jnp.int32, (1, TC), 1).astype(jnp.float32)
            + jnp.float32(TC) * j.astype(jnp.float32))

    @pl.when(j == 0)
    def _():
        A[...] = jnp.full((TR, TC), _F32_MAX, jnp.float32)

    @pl.when(i == 0)
    def _():
        bmin_s[j] = jnp.full((CH, TC), _F32_MAX, jnp.float32)

    rowc = jax.lax.broadcasted_iota(jnp.int32, (CH, 1), 0).astype(jnp.float32)
    ibase = jnp.float32(TR) * i.astype(jnp.float32)

    for r in range(NCH):
        a = ref_blk[r * CH:(r + 1) * CH, :]
        ax = a[:, 0:1]
        ay = a[:, 1:2]
        az = a[:, 2:3]
        dx = ax - bx
        dy = ay - by
        dz = az - bz
        d2 = dx * dx + dy * dy + dz * dz

        sl = slice(r * CH, (r + 1) * CH)
        Ac = A[sl, :]
        m = d2 < Ac
        A[sl, :] = jnp.where(m, d2, Ac)
        I[sl, :] = jnp.where(m, colf, I[sl, :])

        Bv = bmin_s[j]
        m2 = d2 < Bv
        bmin_s[j] = jnp.where(m2, d2, Bv)
        rowids = rowc + (ibase + jnp.float32(r * CH))
        bidx_s[j] = jnp.where(m2, rowids, bidx_s[j])

    @pl.when(j == NJ - 1)
    def _():
        Af = A[...]
        rm = jnp.min(Af, axis=1, keepdims=True)
        cand = jnp.where(Af == rm, I[...], _F32_MAX)
        fwd_min_o[...] = jnp.sqrt(rm)
        fwd_idx_o[...] = jnp.min(cand, axis=1, keepdims=True).astype(jnp.int32)

    @pl.when(i == NI - 1)
    def _():
        Bf = bmin_s[j]
        cm = jnp.min(Bf, axis=0, keepdims=True)
        candr = jnp.where(Bf == cm, bidx_s[j], _F32_MAX)
        bwd_min_o[...] = jnp.sqrt(cm)
        bwd_idx_o[...] = jnp.min(candr, axis=0, keepdims=True).astype(jnp.int32)


def _min_argmin_both(ref_kpts, srcT, gt_transform):
    N = ref_kpts.shape[0]
    M = srcT.shape[1]
    TR = min(512, N)
    TC = min(512, M)
    CH = min(32, TR)
    NI = N // TR
    NJ = M // TC

    body = functools.partial(_dist_body, NI, NJ, TR, TC, CH)
    return pl.pallas_call(
        body,
        grid=(NI, NJ),
        in_specs=[
            pl.BlockSpec((TR, 3), lambda i, j: (i, 0)),
            pl.BlockSpec((3, TC), lambda i, j: (0, j)),
            pl.BlockSpec(memory_space=pltpu.SMEM),
        ],
        out_specs=[
            pl.BlockSpec((TR, 1), lambda i, j: (i, 0)),
            pl.BlockSpec((TR, 1), lambda i, j: (i, 0)),
            pl.BlockSpec((1, TC), lambda i, j: (0, j)),
            pl.BlockSpec((1, TC), lambda i, j: (0, j)),
        ],
        out_shape=[
            jax.ShapeDtypeStruct((N, 1), jnp.float32),
            jax.ShapeDtypeStruct((N, 1), jnp.int32),
            jax.ShapeDtypeStruct((1, M), jnp.float32),
            jax.ShapeDtypeStruct((1, M), jnp.int32),
        ],
        scratch_shapes=[
            pltpu.VMEM((TR, TC), jnp.float32),
            pltpu.VMEM((TR, TC), jnp.float32),
            pltpu.VMEM((NJ, CH, TC), jnp.float32),
            pltpu.VMEM((NJ, CH, TC), jnp.float32),
        ],
        compiler_params=pltpu.CompilerParams(
            dimension_semantics=("arbitrary", "arbitrary"),
        ),
    )(ref_kpts, srcT, gt_transform)


def kernel(ref_kpts, src_kpts, gt_transform, ref_sigma, src_sigma):
    srcT = src_kpts.T
    fm, fi, bm, bi = _min_argmin_both(ref_kpts, srcT, gt_transform)
    fm = fm[:, 0]
    fi = fi[:, 0]
    bm = bm[0]
    bi = bi[0]

    sigma_f = (ref_sigma + jnp.take(src_sigma, fi, axis=0)) * 0.5
    sigma_b = (src_sigma + jnp.take(ref_sigma, bi, axis=0)) * 0.5
    forward_loss = jnp.mean(jnp.log(sigma_f) + fm / sigma_f)
    backward_loss = jnp.mean(jnp.log(sigma_b) + bm / sigma_b)
    return forward_loss + backward_loss


# speedup vs baseline: 1.1297x; 1.1297x over previous
import functools

import jax
import jax.numpy as jnp
from jax.experimental import pallas as pl
from jax.experimental.pallas import tpu as pltpu

_F32_MAX = 3.0e38


def _dist_body(NI, NJ, TR, TC, CH,
               ref_blk, srcT_blk, gt_smem,
               fwd_min_o, fwd_idx_o, bwd_min_o, bwd_idx_o,
               A, I, bmin_s, bidx_s):
    i = pl.program_id(0)
    j = pl.program_id(1)
    NCH = TR // CH

    sx = srcT_blk[0:1, :]
    sy = srcT_blk[1:2, :]
    sz = srcT_blk[2:3, :]
    bx = gt_smem[0, 0] * sx + gt_smem[0, 1] * sy + gt_smem[0, 2] * sz + gt_smem[0, 3]
    by = gt_smem[1, 0] * sx + gt_smem[1, 1] * sy + gt_smem[1, 2] * sz + gt_smem[1, 3]
    bz = gt_smem[2, 0] * sx + gt_smem[2, 1] * sy + gt_smem[2, 2] * sz + gt_smem[2, 3]

    colf = (jax.lax.broadcasted_iota(jnp.int32, (1, TC), 1).astype(jnp.float32)
            + jnp.float32(TC) * j.astype(jnp.float32))

    @pl.when(j == 0)
    def _():
        A[...] = jnp.full((TR, TC), _F32_MAX, jnp.float32)

    @pl.when(i == 0)
    def _():
        bmin_s[j] = jnp.full((CH, TC), _F32_MAX, jnp.float32)

    rowc = jax.lax.broadcasted_iota(jnp.int32, (CH, 1), 0).astype(jnp.float32)
    ibase = jnp.float32(TR) * i.astype(jnp.float32)

    for r in range(NCH):
        a = ref_blk[r * CH:(r + 1) * CH, :]
        ax = a[:, 0:1]
        ay = a[:, 1:2]
        az = a[:, 2:3]
        dx = ax - bx
        dy = ay - by
        dz = az - bz
        d2 = dx * dx + dy * dy + dz * dz

        sl = slice(r * CH, (r + 1) * CH)
        Ac = A[sl, :]
        m = d2 < Ac
        A[sl, :] = jnp.where(m, d2, Ac)
        I[sl, :] = jnp.where(m, colf, I[sl, :])

        Bv = bmin_s[j]
        m2 = d2 < Bv
        bmin_s[j] = jnp.where(m2, d2, Bv)
        rowids = rowc + (ibase + jnp.float32(r * CH))
        bidx_s[j] = jnp.where(m2, rowids, bidx_s[j])

    @pl.when(j == NJ - 1)
    def _():
        Af = A[...]
        rm = jnp.min(Af, axis=1, keepdims=True)
        cand = jnp.where(Af == rm, I[...], _F32_MAX)
        fwd_min_o[...] = jnp.sqrt(rm)
        fwd_idx_o[...] = jnp.min(cand, axis=1, keepdims=True).astype(jnp.int32)

    @pl.when(i == NI - 1)
    def _():
        Bf = bmin_s[j]
        cm = jnp.min(Bf, axis=0, keepdims=True)
        candr = jnp.where(Bf == cm, bidx_s[j], _F32_MAX)
        bwd_min_o[...] = jnp.sqrt(cm)
        bwd_idx_o[...] = jnp.min(candr, axis=0, keepdims=True).astype(jnp.int32)


def _min_argmin_both(ref_kpts, srcT, gt_transform):
    N = ref_kpts.shape[0]
    M = srcT.shape[1]
    TR = min(512, N)
    TC = min(1024, M)
    CH = min(16, TR)
    NI = N // TR
    NJ = M // TC

    body = functools.partial(_dist_body, NI, NJ, TR, TC, CH)
    return pl.pallas_call(
        body,
        grid=(NI, NJ),
        in_specs=[
            pl.BlockSpec((TR, 3), lambda i, j: (i, 0)),
            pl.BlockSpec((3, TC), lambda i, j: (0, j)),
            pl.BlockSpec(memory_space=pltpu.SMEM),
        ],
        out_specs=[
            pl.BlockSpec((TR, 1), lambda i, j: (i, 0)),
            pl.BlockSpec((TR, 1), lambda i, j: (i, 0)),
            pl.BlockSpec((1, TC), lambda i, j: (0, j)),
            pl.BlockSpec((1, TC), lambda i, j: (0, j)),
        ],
        out_shape=[
            jax.ShapeDtypeStruct((N, 1), jnp.float32),
            jax.ShapeDtypeStruct((N, 1), jnp.int32),
            jax.ShapeDtypeStruct((1, M), jnp.float32),
            jax.ShapeDtypeStruct((1, M), jnp.int32),
        ],
        scratch_shapes=[
            pltpu.VMEM((TR, TC), jnp.float32),
            pltpu.VMEM((TR, TC), jnp.float32),
            pltpu.VMEM((NJ, CH, TC), jnp.float32),
            pltpu.VMEM((NJ, CH, TC), jnp.float32),
        ],
        compiler_params=pltpu.CompilerParams(
            dimension_semantics=("arbitrary", "arbitrary"),
        ),
    )(ref_kpts, srcT, gt_transform)


def kernel(ref_kpts, src_kpts, gt_transform, ref_sigma, src_sigma):
    srcT = src_kpts.T
    fm, fi, bm, bi = _min_argmin_both(ref_kpts, srcT, gt_transform)
    fm = fm[:, 0]
    fi = fi[:, 0]
    bm = bm[0]
    bi = bi[0]

    sigma_f = (ref_sigma + jnp.take(src_sigma, fi, axis=0)) * 0.5
    sigma_b = (src_sigma + jnp.take(ref_sigma, bi, axis=0)) * 0.5
    forward_loss = jnp.mean(jnp.log(sigma_f) + fm / sigma_f)
    backward_loss = jnp.mean(jnp.log(sigma_b) + bm / sigma_b)
    return forward_loss + backward_loss
